# Initial kernel scaffold; baseline (speedup 1.0000x reference)
import jax
import jax.numpy as jnp
from jax.experimental import pallas as pl


def kernel(anchors, deltas, scores):
    raise NotImplementedError("write your pallas kernel here")


# speedup vs baseline: 17.8250x; 17.8250x over previous
import math

import jax
import jax.numpy as jnp
from jax.experimental import pallas as pl

_PRE = 6000
_POST = 300
_THRESH = 0.7
_ROWS = 48
_PAD = _ROWS * 128
_NEG = -1e9
_LOG_MAX_RATIO = math.log(1000.0 / 16.0)


def _nms_body(a_ref, d_ref, s_ref, o_ref):
    ax1, ay1, ax2, ay2 = a_ref[0], a_ref[1], a_ref[2], a_ref[3]
    dx, dy, dw, dh = d_ref[0], d_ref[1], d_ref[2], d_ref[3]

    widths = ax2 - ax1 + 1.0
    heights = ay2 - ay1 + 1.0
    ctr_x = ax1 + 0.5 * widths
    ctr_y = ay1 + 0.5 * heights
    dw = jnp.minimum(dw, _LOG_MAX_RATIO)
    dh = jnp.minimum(dh, _LOG_MAX_RATIO)
    pcx = dx * widths + ctr_x
    pcy = dy * heights + ctr_y
    pw = jnp.exp(dw) * widths
    ph = jnp.exp(dh) * heights
    x1 = jnp.clip(pcx - 0.5 * pw, 0.0, 1023.0)
    y1 = jnp.clip(pcy - 0.5 * ph, 0.0, 1023.0)
    x2 = jnp.clip(pcx + 0.5 * pw, 0.0, 1023.0)
    y2 = jnp.clip(pcy + 0.5 * ph, 0.0, 1023.0)
    areas = (x2 - x1 + 1.0) * (y2 - y1 + 1.0)
    scores = s_ref[...]

    lin = (jax.lax.broadcasted_iota(jnp.int32, (_ROWS, 128), 0) * 128
           + jax.lax.broadcasted_iota(jnp.int32, (_ROWS, 128), 1))
    lane = jax.lax.broadcasted_iota(jnp.int32, (1, 128), 1)
    big = jnp.int32(2 ** 30)

    def body(i, work):
        cand = jnp.where(work > jnp.float32(-5e8), lin, big)
        first = jnp.min(cand)
        valid = first < big
        sel = lin == first
        sf = sel.astype(jnp.float32)
        bx1 = jnp.sum(sf * x1)
        by1 = jnp.sum(sf * y1)
        bx2 = jnp.sum(sf * x2)
        by2 = jnp.sum(sf * y2)
        bsc = jnp.sum(sf * scores)
        bar = jnp.sum(sf * areas)
        xx1 = jnp.maximum(bx1, x1)
        yy1 = jnp.maximum(by1, y1)
        xx2 = jnp.minimum(bx2, x2)
        yy2 = jnp.minimum(by2, y2)
        w = jnp.maximum(0.0, xx2 - xx1 + 1.0)
        h = jnp.maximum(0.0, yy2 - yy1 + 1.0)
        inter = w * h
        iou = inter / (bar + areas - inter)
        supp = jnp.logical_and(valid, jnp.logical_or(iou >= _THRESH, sel))
        new_work = jnp.where(supp, jnp.float32(_NEG), work)
        vf = jnp.where(valid, jnp.float32(1.0), jnp.float32(0.0))
        row = (jnp.where(lane == 0, bx1, 0.0)
               + jnp.where(lane == 1, by1, 0.0)
               + jnp.where(lane == 2, bx2, 0.0)
               + jnp.where(lane == 3, by2, 0.0)
               + jnp.where(lane == 4, bsc, 0.0)) * vf
        o_ref[pl.ds(i, 1), :] = row
        return new_work

    jax.lax.fori_loop(0, _POST, body, scores)


def _prep(anchors, deltas, scores):
    top_scores, order = jax.lax.top_k(scores, _PRE)
    a = jnp.take(anchors, order, axis=0)
    d = jnp.take(deltas, order, axis=0)
    a_t = jnp.pad(a, ((0, _PAD - _PRE), (0, 0))).T.reshape(4, _ROWS, 128)
    d_t = jnp.pad(d, ((0, _PAD - _PRE), (0, 0))).T.reshape(4, _ROWS, 128)
    s_t = jnp.pad(top_scores, (0, _PAD - _PRE),
                  constant_values=-1e30).reshape(_ROWS, 128)
    return a_t, d_t, s_t


@jax.jit
def kernel(anchors, deltas, scores):
    a_t, d_t, s_t = _prep(anchors, deltas, scores)
    out = pl.pallas_call(
        _nms_body,
        out_shape=jax.ShapeDtypeStruct((304, 128), jnp.float32),
    )(a_t, d_t, s_t)
    return out[:300, :5]
